# Initial kernel scaffold; baseline (speedup 1.0000x reference)
import jax
import jax.numpy as jnp
from jax.experimental import pallas as pl


def kernel(h_v, h_p, h_t, interacts_src, interacts_dst, agg_src, agg_dst, W_pi, W_q, W_r):
    raise NotImplementedError("write your pallas kernel here")


# speedup vs baseline: 6.4244x; 6.4244x over previous
import functools

import jax
import jax.numpy as jnp
import numpy as np
from jax import lax
from jax.experimental import pallas as pl
from jax.experimental.pallas import tpu as pltpu
from jax.experimental.pallas import tpu_sc as plsc

DIM = 128
NI = 10000
NIP = 10240
NT = 4096
EI = 320000
EA = 81920

NC = 2
NS = 16
NW = NC * NS
L = 16

K = 80
SR = 128

_MESH = plsc.VectorSubcoreMesh(core_axis_name="c", subcore_axis_name="s",
                               num_cores=NC, num_subcores=NS)


def _hsum(v):
    s = v[0]
    for j in range(1, L):
        s = s + v[j]
    return s


@functools.partial(
    pl.kernel,
    out_type=(jax.ShapeDtypeStruct((NC, NIP, DIM), jnp.float32),
              jax.ShapeDtypeStruct((NC, SR, DIM), jnp.float32)),
    mesh=_MESH,
    scratch_types=[
        pltpu.VMEM((K,), jnp.int32),
        pltpu.VMEM((K,), jnp.int32),
        pltpu.VMEM((K,), jnp.int32),
        pltpu.VMEM((K, DIM), jnp.float32),
        pltpu.VMEM((K, DIM), jnp.float32),
        pltpu.VMEM((K, DIM), jnp.float32),
        pltpu.VMEM((K, DIM), jnp.float32),
        pltpu.VMEM((K,), jnp.float32),
        pltpu.VMEM((DIM,), jnp.float32),
        pltpu.VMEM_SHARED((NIP, DIM), jnp.float32),
        pltpu.VMEM_SHARED((SR, DIM), jnp.float32),
        pltpu.SemaphoreType.DMA,
        pltpu.SemaphoreType.DMA,
    ],
)
def _p1(h_v, i_src, i_dst, w, zf, zs, part, spart,
        isrc_v, idst_v, sidx_v, srcr, dstr, msg, smsg, ebuf, w_v,
        ft_sh, s2_sh, sem0, sem1):
    cid = lax.axis_index("c")
    sid = lax.axis_index("s")
    wid = cid * NS + sid
    rpt = NIP // NS

    pltpu.sync_copy(zf.at[pl.ds(sid * rpt, rpt)],
                    ft_sh.at[pl.ds(sid * rpt, rpt)])
    pltpu.sync_copy(zs.at[pl.ds(sid * 8, 8)], s2_sh.at[pl.ds(sid * 8, 8)])
    pltpu.sync_copy(w, w_v)
    plsc.subcore_barrier()

    wb = [w_v[pl.ds(b * L, L)] for b in range(DIM // L)]
    lanes = lax.iota(jnp.int32, L)

    def chunk_body(c, _):
        base = wid * (EI // NW) + c * K
        pltpu.sync_copy(i_src.at[pl.ds(base, K)], isrc_v)
        pltpu.sync_copy(i_dst.at[pl.ds(base, K)], idst_v)
        cp0 = pltpu.async_copy(h_v.at[isrc_v], srcr, sem0)
        cp1 = pltpu.async_copy(h_v.at[idst_v], dstr, sem1)
        cp0.wait()
        cp1.wait()

        def e_body(g, _):
            ev = jnp.zeros((L,), jnp.float32)
            for j in range(L):
                k = g * L + j
                acc = srcr[k, pl.ds(0, L)] * dstr[k, pl.ds(0, L)] * wb[0]
                for b in range(1, DIM // L):
                    acc = acc + srcr[k, pl.ds(b * L, L)] * dstr[k, pl.ds(b * L, L)] * wb[b]
                ev = jnp.where(lanes == j, _hsum(acc), ev)
            ebuf[pl.ds(g * L, L)] = jnp.exp(ev)
            idg = idst_v[pl.ds(g * L, L)]
            sidx_v[pl.ds(g * L, L)] = lax.shift_right_logical(idg, 7)
            return 0
        lax.fori_loop(0, K // L, e_body, 0)

        def m_body(g, _):
            exv = ebuf[pl.ds(g * L, L)]
            idg = idst_v[pl.ds(g * L, L)]
            for j in range(L):
                k = g * L + j
                ex = exv[j]
                col = idg[j] & (DIM - 1)
                for b in range(DIM // L):
                    msg[k, pl.ds(b * L, L)] = srcr[k, pl.ds(b * L, L)] * ex
                    smsg[k, pl.ds(b * L, L)] = jnp.where(lanes == col - b * L, ex, 0.0)
            return 0
        lax.fori_loop(0, K // L, m_body, 0)

        pltpu.sync_copy(msg, ft_sh.at[idst_v], add=True)
        pltpu.sync_copy(smsg, s2_sh.at[sidx_v], add=True)
        return 0

    lax.fori_loop(0, EI // NW // K, chunk_body, 0)
    plsc.subcore_barrier()
    pltpu.sync_copy(ft_sh.at[pl.ds(sid * rpt, rpt)],
                    part.at[cid, pl.ds(sid * rpt, rpt)])
    pltpu.sync_copy(s2_sh.at[pl.ds(sid * 8, 8)],
                    spart.at[cid, pl.ds(sid * 8, 8)])


def _n_body(part_ref, s_ref, out_ref):
    num = part_ref[0] + part_ref[1]
    out_ref[...] = num / (s_ref[...] + 1e-16)


def _n(part, s_col):
    blk = 1024
    return pl.pallas_call(
        _n_body,
        grid=(NIP // blk,),
        in_specs=[pl.BlockSpec((NC, blk, DIM), lambda i: (0, i, 0)),
                  pl.BlockSpec((blk, 1), lambda i: (i, 0))],
        out_specs=pl.BlockSpec((blk, DIM), lambda i: (i, 0)),
        out_shape=jax.ShapeDtypeStruct((NIP, DIM), jnp.float32),
    )(part, s_col)


@functools.partial(
    pl.kernel,
    out_type=(jax.ShapeDtypeStruct((EA, DIM), jnp.float32),
              jax.ShapeDtypeStruct((NC, NT, DIM), jnp.float32),
              jax.ShapeDtypeStruct((NC, SR, DIM), jnp.float32)),
    mesh=_MESH,
    scratch_types=[
        pltpu.VMEM((K,), jnp.int32),
        pltpu.VMEM((K,), jnp.int32),
        pltpu.VMEM((K,), jnp.int32),
        pltpu.VMEM((K, DIM), jnp.float32),
        pltpu.VMEM((K, DIM), jnp.float32),
        pltpu.VMEM_SHARED((NT, DIM), jnp.float32),
        pltpu.VMEM_SHARED((SR, DIM), jnp.float32),
        pltpu.SemaphoreType.DMA,
    ],
)
def _b(ft_item, a_src, a_dst, zm, zs, edge_ft, mparts, dparts,
       asrc_v, adst_v, didx_v, rows, dmsg, mean_sh, deg_sh, sem0):
    cid = lax.axis_index("c")
    sid = lax.axis_index("s")
    wid = cid * NS + sid
    rpt = NT // NS

    pltpu.sync_copy(zm.at[pl.ds(sid * rpt, rpt)], mean_sh.at[pl.ds(sid * rpt, rpt)])
    pltpu.sync_copy(zs.at[pl.ds(sid * 8, 8)], deg_sh.at[pl.ds(sid * 8, 8)])
    plsc.subcore_barrier()

    lanes = lax.iota(jnp.int32, L)

    def chunk_body(c, _):
        base = wid * (EA // NW) + c * K
        pltpu.sync_copy(a_src.at[pl.ds(base, K)], asrc_v)
        pltpu.sync_copy(a_dst.at[pl.ds(base, K)], adst_v)
        pltpu.async_copy(ft_item.at[asrc_v], rows, sem0).wait()
        pltpu.sync_copy(rows, edge_ft.at[pl.ds(base, K)])
        pltpu.sync_copy(rows, mean_sh.at[adst_v], add=True)

        def d_body(g, _):
            idg = adst_v[pl.ds(g * L, L)]
            didx_v[pl.ds(g * L, L)] = lax.shift_right_logical(idg, 7)
            for j in range(L):
                k = g * L + j
                col = idg[j] & (DIM - 1)
                for b in range(DIM // L):
                    dmsg[k, pl.ds(b * L, L)] = jnp.where(lanes == col - b * L, 1.0, 0.0)
            return 0
        lax.fori_loop(0, K // L, d_body, 0)

        pltpu.sync_copy(dmsg, deg_sh.at[didx_v], add=True)
        return 0

    lax.fori_loop(0, EA // NW // K, chunk_body, 0)
    plsc.subcore_barrier()
    pltpu.sync_copy(mean_sh.at[pl.ds(sid * rpt, rpt)],
                    mparts.at[cid, pl.ds(sid * rpt, rpt)])
    pltpu.sync_copy(deg_sh.at[pl.ds(sid * 8, 8)],
                    dparts.at[cid, pl.ds(sid * 8, 8)])


def _m1_body(mp_ref, deg_ref, ht_ref, wr_ref, out_ref):
    mean = (mp_ref[0] + mp_ref[1]) / jnp.maximum(deg_ref[...], 1.0)
    out_ref[...] = (
        jnp.dot(ht_ref[...], wr_ref[:DIM], preferred_element_type=jnp.float32)
        + jnp.dot(mean, wr_ref[DIM:], preferred_element_type=jnp.float32))


def _m1(mparts, deg_col, h_t, W_r):
    blk = 1024
    return pl.pallas_call(
        _m1_body,
        grid=(NT // blk,),
        in_specs=[pl.BlockSpec((NC, blk, DIM), lambda i: (0, i, 0)),
                  pl.BlockSpec((blk, 1), lambda i: (i, 0)),
                  pl.BlockSpec((blk, DIM), lambda i: (i, 0)),
                  pl.BlockSpec((2 * DIM, DIM), lambda i: (0, 0))],
        out_specs=pl.BlockSpec((blk, DIM), lambda i: (i, 0)),
        out_shape=jax.ShapeDtypeStruct((NT, DIM), jnp.float32),
    )(mparts, deg_col, h_t, W_r)


def _m2_body(eft_ref, hp_ref, wq_ref, out_ref):
    out_ref[...] = jnp.tanh(
        jnp.dot(eft_ref[...], wq_ref[:DIM], preferred_element_type=jnp.float32)
        + jnp.dot(hp_ref[...], wq_ref[DIM:], preferred_element_type=jnp.float32))


def _m2(edge_ft, h_p, W_q):
    blk = 1024
    return pl.pallas_call(
        _m2_body,
        grid=(EA // blk,),
        in_specs=[pl.BlockSpec((blk, DIM), lambda i: (i, 0)),
                  pl.BlockSpec((blk, DIM), lambda i: (i, 0)),
                  pl.BlockSpec((2 * DIM, DIM), lambda i: (0, 0))],
        out_specs=pl.BlockSpec((blk, DIM), lambda i: (i, 0)),
        out_shape=jax.ShapeDtypeStruct((EA, DIM), jnp.float32),
    )(edge_ft, h_p, W_q)


@functools.partial(
    pl.kernel,
    out_type=jax.ShapeDtypeStruct((NC, NT, DIM), jnp.float32),
    mesh=_MESH,
    scratch_types=[
        pltpu.VMEM((K,), jnp.int32),
        pltpu.VMEM((K, DIM), jnp.float32),
        pltpu.VMEM((K, DIM), jnp.float32),
        pltpu.VMEM((K, DIM), jnp.float32),
        pltpu.VMEM((K, DIM), jnp.float32),
        pltpu.VMEM((K,), jnp.float32),
        pltpu.VMEM_SHARED((NT, DIM), jnp.float32),
        pltpu.SemaphoreType.DMA,
    ],
)
def _c(edge_ft, e2, f, a_dst, zo, oparts,
       adst_v, eftr, e2r, fr, msg, cbuf, out_sh, sem0):
    cid = lax.axis_index("c")
    sid = lax.axis_index("s")
    wid = cid * NS + sid
    rpt = NT // NS

    pltpu.sync_copy(zo.at[pl.ds(sid * rpt, rpt)], out_sh.at[pl.ds(sid * rpt, rpt)])
    plsc.subcore_barrier()

    lanes = lax.iota(jnp.int32, L)

    def chunk_body(c, _):
        base = wid * (EA // NW) + c * K
        pltpu.sync_copy(a_dst.at[pl.ds(base, K)], adst_v)
        pltpu.sync_copy(edge_ft.at[pl.ds(base, K)], eftr)
        pltpu.sync_copy(e2.at[pl.ds(base, K)], e2r)
        pltpu.async_copy(f.at[adst_v], fr, sem0).wait()

        def d_body(g, _):
            cv = jnp.zeros((L,), jnp.float32)
            for j in range(L):
                k = g * L + j
                acc = e2r[k, pl.ds(0, L)] * fr[k, pl.ds(0, L)]
                for b in range(1, DIM // L):
                    acc = acc + e2r[k, pl.ds(b * L, L)] * fr[k, pl.ds(b * L, L)]
                cv = jnp.where(lanes == j, _hsum(acc), cv)
            cbuf[pl.ds(g * L, L)] = cv
            return 0
        lax.fori_loop(0, K // L, d_body, 0)

        def m_body(g, _):
            csv = cbuf[pl.ds(g * L, L)]
            for j in range(L):
                k = g * L + j
                cs = csv[j]
                for b in range(DIM // L):
                    msg[k, pl.ds(b * L, L)] = eftr[k, pl.ds(b * L, L)] * cs
            return 0
        lax.fori_loop(0, K // L, m_body, 0)

        pltpu.sync_copy(msg, out_sh.at[adst_v], add=True)
        return 0

    lax.fori_loop(0, EA // NW // K, chunk_body, 0)
    plsc.subcore_barrier()
    pltpu.sync_copy(out_sh.at[pl.ds(sid * rpt, rpt)],
                    oparts.at[cid, pl.ds(sid * rpt, rpt)])


def _f_body(op_ref, out_ref):
    out_ref[...] = op_ref[0] + op_ref[1]


def _f(oparts):
    blk = 1024
    return pl.pallas_call(
        _f_body,
        grid=(NT // blk,),
        in_specs=[pl.BlockSpec((NC, blk, DIM), lambda i: (0, i, 0))],
        out_specs=pl.BlockSpec((blk, DIM), lambda i: (i, 0)),
        out_shape=jax.ShapeDtypeStruct((NT, DIM), jnp.float32),
    )(oparts)


def kernel(h_v, h_p, h_t, interacts_src, interacts_dst, agg_src, agg_dst,
           W_pi, W_q, W_r):
    i_src = interacts_src.astype(jnp.int32)
    i_dst = interacts_dst.astype(jnp.int32)
    a_src = agg_src.astype(jnp.int32)
    a_dst = agg_dst.astype(jnp.int32)
    w = W_pi.reshape(DIM)

    zf = jnp.zeros((NIP, DIM), jnp.float32)
    zm = jnp.zeros((NT, DIM), jnp.float32)
    zs = jnp.zeros((SR, DIM), jnp.float32)

    part, spart = _p1(h_v, i_src, i_dst, w, zf, zs)
    s_col = (spart[0] + spart[1]).reshape(SR * DIM)[:NIP].reshape(NIP, 1)
    ft_item = _n(part, s_col)
    edge_ft, mparts, dparts = _b(ft_item, a_src, a_dst, zm, zs)
    deg_col = (dparts[0] + dparts[1]).reshape(SR * DIM)[:NT].reshape(NT, 1)
    f = _m1(mparts, deg_col, h_t, W_r)
    e2 = _m2(edge_ft, h_p, W_q)
    oparts = _c(edge_ft, e2, f, a_dst, zm)
    return _f(oparts)


# speedup vs baseline: 8.2070x; 1.2775x over previous
import functools

import jax
import jax.numpy as jnp
import numpy as np
from jax import lax
from jax.experimental import pallas as pl
from jax.experimental.pallas import tpu as pltpu
from jax.experimental.pallas import tpu_sc as plsc

DIM = 128
NI = 10000
NIP = 10240
NT = 4096
EI = 320000
EA = 81920

NC = 2
NS = 16
NW = NC * NS
L = 16

K = 80
SR = 128

_MESH = plsc.VectorSubcoreMesh(core_axis_name="c", subcore_axis_name="s",
                               num_cores=NC, num_subcores=NS)


def _hsum(v):
    s = v[0]
    for j in range(1, L):
        s = s + v[j]
    return s


_P1_NCHUNK = EI // NW // K
_P1_SCRATCH = []
for _ in range(2):
    _P1_SCRATCH += [
        pltpu.VMEM((K,), jnp.int32),
        pltpu.VMEM((K,), jnp.int32),
        pltpu.VMEM((K,), jnp.int32),
        pltpu.VMEM((K, DIM), jnp.float32),
        pltpu.VMEM((K,), jnp.float32),
        pltpu.SemaphoreType.DMA,
        pltpu.SemaphoreType.DMA,
        pltpu.SemaphoreType.DMA,
    ]
_P1_SCRATCH += [
    pltpu.VMEM((K, DIM), jnp.float32),
    pltpu.SemaphoreType.DMA,
    pltpu.VMEM((NIP,), jnp.float32),
    pltpu.VMEM((DIM,), jnp.float32),
    pltpu.VMEM_SHARED((NIP, DIM), jnp.float32),
]


@functools.partial(
    pl.kernel,
    out_type=(jax.ShapeDtypeStruct((NC, NIP, DIM), jnp.float32),
              jax.ShapeDtypeStruct((NW * NIP,), jnp.float32)),
    mesh=_MESH,
    scratch_types=_P1_SCRATCH,
    compiler_params=pltpu.CompilerParams(needs_layout_passes=False),
)
def _p1(h_v, i_src, i_dst, w, zf, part, sparts, *scr):
    sets = [dict(zip(("isrc", "idst", "fidx", "srcr", "ebuf",
                      "semi", "semsrc", "semsc"), scr[i*8:(i+1)*8]))
            for i in range(2)]
    dstr, semdst, s_tab, w_v, ft_sh = scr[16:21]
    cid = lax.axis_index("c")
    sid = lax.axis_index("s")
    wid = cid * NS + sid
    rpt = NIP // NS
    edge0 = wid * (EI // NW)

    pltpu.sync_copy(zf.at[pl.ds(sid * rpt, rpt)],
                    ft_sh.at[pl.ds(sid * rpt, rpt)])
    pltpu.sync_copy(w, w_v)
    zv = jnp.zeros((L,), jnp.float32)

    def z_body(i, _):
        s_tab[pl.ds(i * L, L)] = zv
        return 0
    lax.fori_loop(0, NIP // L, z_body, 0)
    plsc.subcore_barrier()

    wb = [w_v[pl.ds(b * L, L)] for b in range(DIM // L)]
    lanes = lax.iota(jnp.int32, L)

    def start_idx(c, S):
        base = edge0 + c * K
        pltpu.async_copy(i_src.at[pl.ds(base, K)], S["isrc"], S["semi"])
        pltpu.async_copy(i_dst.at[pl.ds(base, K)], S["idst"], S["semi"])

    def wait_idx(S):
        pltpu.make_async_copy(i_src.at[pl.ds(0, K)], S["isrc"], S["semi"]).wait()
        pltpu.make_async_copy(i_dst.at[pl.ds(0, K)], S["idst"], S["semi"]).wait()

    def start_src(S):
        pltpu.async_copy(h_v.at[S["isrc"]], S["srcr"], S["semsrc"])

    def wait_src(S):
        pltpu.make_async_copy(h_v.at[S["isrc"]], S["srcr"], S["semsrc"]).wait()

    def start_dst(S):
        pltpu.async_copy(h_v.at[S["idst"]], dstr, semdst)

    def wait_dst(S):
        pltpu.make_async_copy(h_v.at[S["idst"]], dstr, semdst).wait()

    def start_scatter(S):
        pltpu.async_copy(S["srcr"], ft_sh.at[S["fidx"]], S["semsc"], add=True)

    def wait_scatter(S):
        pltpu.make_async_copy(S["srcr"], ft_sh.at[S["fidx"]], S["semsc"]).wait()

    def e_pass(S):
        srcr, ebuf = S["srcr"], S["ebuf"]

        def e_body(g, _):
            ev = jnp.zeros((L,), jnp.float32)
            for j in range(L):
                k = g * L + j
                acc = srcr[k, pl.ds(0, L)] * dstr[k, pl.ds(0, L)] * wb[0]
                for b in range(1, DIM // L):
                    acc = acc + srcr[k, pl.ds(b * L, L)] * dstr[k, pl.ds(b * L, L)] * wb[b]
                ev = jnp.where(lanes == j, _hsum(acc), ev)
            ebuf[pl.ds(g * L, L)] = jnp.exp(ev)
            S["fidx"][pl.ds(g * L, L)] = S["idst"][pl.ds(g * L, L)]
            return 0
        lax.fori_loop(0, K // L, e_body, 0)

    def m_pass(S):
        srcr, ebuf = S["srcr"], S["ebuf"]

        def m_body(g, _):
            exv = ebuf[pl.ds(g * L, L)]
            idg = S["fidx"][pl.ds(g * L, L)]
            for j in range(L):
                k = g * L + j
                ex = exv[j]
                for b in range(DIM // L):
                    srcr[k, pl.ds(b * L, L)] = srcr[k, pl.ds(b * L, L)] * ex
                plsc.addupdate_scatter(s_tab, [idg], exv, mask=lanes == j)
            return 0
        lax.fori_loop(0, K // L, m_body, 0)

    def body(c, s, tail=False, first=False, guard_idx=None):
        S, S2 = sets[s], sets[1 - s]
        wait_src(S)
        wait_dst(S)
        e_pass(S)
        if not tail:
            wait_idx(S2)
            start_dst(S2)
            if first:
                pass
            else:
                wait_scatter(S2)
            start_src(S2)
        m_pass(S)
        start_scatter(S)
        if not tail:
            if guard_idx is None:
                start_idx(c + 2, S)
            else:
                @pl.when(guard_idx)
                def _():
                    start_idx(c + 2, S)

    start_idx(0, sets[0])
    start_idx(1, sets[1])
    wait_idx(sets[0])
    start_src(sets[0])
    start_dst(sets[0])

    npair = (_P1_NCHUNK - 1) // 2

    def loop_first(_, __):
        body(0, 0, first=True)
        body(1, 1)
        return 0
    lax.fori_loop(0, 1, loop_first, 0)

    def loop_body(i, _):
        c = (i + 1) * 2
        body(c, 0)
        body(c + 1, 1, guard_idx=(i < npair - 2))
        return 0
    lax.fori_loop(0, npair - 1, loop_body, 0)

    body(_P1_NCHUNK - 1, 0, tail=True)
    wait_scatter(sets[0])
    wait_scatter(sets[1])

    plsc.subcore_barrier()
    pltpu.sync_copy(ft_sh.at[pl.ds(sid * rpt, rpt)],
                    part.at[cid, pl.ds(sid * rpt, rpt)])
    pltpu.sync_copy(s_tab, sparts.at[pl.ds(wid * NIP, NIP)])


def _n_body(part_ref, s_ref, out_ref):
    num = part_ref[0] + part_ref[1]
    out_ref[...] = num / (s_ref[...] + 1e-16)


def _n(part, s_col):
    blk = 1024
    return pl.pallas_call(
        _n_body,
        grid=(NIP // blk,),
        in_specs=[pl.BlockSpec((NC, blk, DIM), lambda i: (0, i, 0)),
                  pl.BlockSpec((blk, 1), lambda i: (i, 0))],
        out_specs=pl.BlockSpec((blk, DIM), lambda i: (i, 0)),
        out_shape=jax.ShapeDtypeStruct((NIP, DIM), jnp.float32),
    )(part, s_col)


@functools.partial(
    pl.kernel,
    out_type=(jax.ShapeDtypeStruct((EA, DIM), jnp.float32),
              jax.ShapeDtypeStruct((NC, NT, DIM), jnp.float32),
              jax.ShapeDtypeStruct((NC, SR, DIM), jnp.float32)),
    mesh=_MESH,
    scratch_types=[
        pltpu.VMEM((K,), jnp.int32),
        pltpu.VMEM((K,), jnp.int32),
        pltpu.VMEM((K,), jnp.int32),
        pltpu.VMEM((K, DIM), jnp.float32),
        pltpu.VMEM((K, DIM), jnp.float32),
        pltpu.VMEM_SHARED((NT, DIM), jnp.float32),
        pltpu.VMEM_SHARED((SR, DIM), jnp.float32),
        pltpu.SemaphoreType.DMA,
    ],
    compiler_params=pltpu.CompilerParams(needs_layout_passes=False),
)
def _b(ft_item, a_src, a_dst, zm, zs, edge_ft, mparts, dparts,
       asrc_v, adst_v, didx_v, rows, dmsg, mean_sh, deg_sh, sem0):
    cid = lax.axis_index("c")
    sid = lax.axis_index("s")
    wid = cid * NS + sid
    rpt = NT // NS

    pltpu.sync_copy(zm.at[pl.ds(sid * rpt, rpt)], mean_sh.at[pl.ds(sid * rpt, rpt)])
    pltpu.sync_copy(zs.at[pl.ds(sid * 8, 8)], deg_sh.at[pl.ds(sid * 8, 8)])
    plsc.subcore_barrier()

    lanes = lax.iota(jnp.int32, L)

    def chunk_body(c, _):
        base = wid * (EA // NW) + c * K
        pltpu.sync_copy(a_src.at[pl.ds(base, K)], asrc_v)
        pltpu.sync_copy(a_dst.at[pl.ds(base, K)], adst_v)
        pltpu.async_copy(ft_item.at[asrc_v], rows, sem0).wait()
        pltpu.sync_copy(rows, edge_ft.at[pl.ds(base, K)])
        pltpu.sync_copy(rows, mean_sh.at[adst_v], add=True)

        def d_body(g, _):
            idg = adst_v[pl.ds(g * L, L)]
            didx_v[pl.ds(g * L, L)] = lax.shift_right_logical(idg, 7)
            for j in range(L):
                k = g * L + j
                col = idg[j] & (DIM - 1)
                for b in range(DIM // L):
                    dmsg[k, pl.ds(b * L, L)] = jnp.where(lanes == col - b * L, 1.0, 0.0)
            return 0
        lax.fori_loop(0, K // L, d_body, 0)

        pltpu.sync_copy(dmsg, deg_sh.at[didx_v], add=True)
        return 0

    lax.fori_loop(0, EA // NW // K, chunk_body, 0)
    plsc.subcore_barrier()
    pltpu.sync_copy(mean_sh.at[pl.ds(sid * rpt, rpt)],
                    mparts.at[cid, pl.ds(sid * rpt, rpt)])
    pltpu.sync_copy(deg_sh.at[pl.ds(sid * 8, 8)],
                    dparts.at[cid, pl.ds(sid * 8, 8)])


def _m1_body(mp_ref, deg_ref, ht_ref, wr_ref, out_ref):
    mean = (mp_ref[0] + mp_ref[1]) / jnp.maximum(deg_ref[...], 1.0)
    out_ref[...] = (
        jnp.dot(ht_ref[...], wr_ref[:DIM], preferred_element_type=jnp.float32)
        + jnp.dot(mean, wr_ref[DIM:], preferred_element_type=jnp.float32))


def _m1(mparts, deg_col, h_t, W_r):
    blk = 1024
    return pl.pallas_call(
        _m1_body,
        grid=(NT // blk,),
        in_specs=[pl.BlockSpec((NC, blk, DIM), lambda i: (0, i, 0)),
                  pl.BlockSpec((blk, 1), lambda i: (i, 0)),
                  pl.BlockSpec((blk, DIM), lambda i: (i, 0)),
                  pl.BlockSpec((2 * DIM, DIM), lambda i: (0, 0))],
        out_specs=pl.BlockSpec((blk, DIM), lambda i: (i, 0)),
        out_shape=jax.ShapeDtypeStruct((NT, DIM), jnp.float32),
    )(mparts, deg_col, h_t, W_r)


def _m2_body(eft_ref, hp_ref, wq_ref, out_ref):
    out_ref[...] = jnp.tanh(
        jnp.dot(eft_ref[...], wq_ref[:DIM], preferred_element_type=jnp.float32)
        + jnp.dot(hp_ref[...], wq_ref[DIM:], preferred_element_type=jnp.float32))


def _m2(edge_ft, h_p, W_q):
    blk = 1024
    return pl.pallas_call(
        _m2_body,
        grid=(EA // blk,),
        in_specs=[pl.BlockSpec((blk, DIM), lambda i: (i, 0)),
                  pl.BlockSpec((blk, DIM), lambda i: (i, 0)),
                  pl.BlockSpec((2 * DIM, DIM), lambda i: (0, 0))],
        out_specs=pl.BlockSpec((blk, DIM), lambda i: (i, 0)),
        out_shape=jax.ShapeDtypeStruct((EA, DIM), jnp.float32),
    )(edge_ft, h_p, W_q)


@functools.partial(
    pl.kernel,
    out_type=jax.ShapeDtypeStruct((NC, NT, DIM), jnp.float32),
    mesh=_MESH,
    scratch_types=[
        pltpu.VMEM((K,), jnp.int32),
        pltpu.VMEM((K, DIM), jnp.float32),
        pltpu.VMEM((K, DIM), jnp.float32),
        pltpu.VMEM((K, DIM), jnp.float32),
        pltpu.VMEM((K, DIM), jnp.float32),
        pltpu.VMEM((K,), jnp.float32),
        pltpu.VMEM_SHARED((NT, DIM), jnp.float32),
        pltpu.SemaphoreType.DMA,
    ],
    compiler_params=pltpu.CompilerParams(needs_layout_passes=False),
)
def _c(edge_ft, e2, f, a_dst, zo, oparts,
       adst_v, eftr, e2r, fr, msg, cbuf, out_sh, sem0):
    cid = lax.axis_index("c")
    sid = lax.axis_index("s")
    wid = cid * NS + sid
    rpt = NT // NS

    pltpu.sync_copy(zo.at[pl.ds(sid * rpt, rpt)], out_sh.at[pl.ds(sid * rpt, rpt)])
    plsc.subcore_barrier()

    lanes = lax.iota(jnp.int32, L)

    def chunk_body(c, _):
        base = wid * (EA // NW) + c * K
        pltpu.sync_copy(a_dst.at[pl.ds(base, K)], adst_v)
        pltpu.sync_copy(edge_ft.at[pl.ds(base, K)], eftr)
        pltpu.sync_copy(e2.at[pl.ds(base, K)], e2r)
        pltpu.async_copy(f.at[adst_v], fr, sem0).wait()

        def d_body(g, _):
            cv = jnp.zeros((L,), jnp.float32)
            for j in range(L):
                k = g * L + j
                acc = e2r[k, pl.ds(0, L)] * fr[k, pl.ds(0, L)]
                for b in range(1, DIM // L):
                    acc = acc + e2r[k, pl.ds(b * L, L)] * fr[k, pl.ds(b * L, L)]
                cv = jnp.where(lanes == j, _hsum(acc), cv)
            cbuf[pl.ds(g * L, L)] = cv
            return 0
        lax.fori_loop(0, K // L, d_body, 0)

        def m_body(g, _):
            csv = cbuf[pl.ds(g * L, L)]
            for j in range(L):
                k = g * L + j
                cs = csv[j]
                for b in range(DIM // L):
                    msg[k, pl.ds(b * L, L)] = eftr[k, pl.ds(b * L, L)] * cs
            return 0
        lax.fori_loop(0, K // L, m_body, 0)

        pltpu.sync_copy(msg, out_sh.at[adst_v], add=True)
        return 0

    lax.fori_loop(0, EA // NW // K, chunk_body, 0)
    plsc.subcore_barrier()
    pltpu.sync_copy(out_sh.at[pl.ds(sid * rpt, rpt)],
                    oparts.at[cid, pl.ds(sid * rpt, rpt)])


def _f_body(op_ref, out_ref):
    out_ref[...] = op_ref[0] + op_ref[1]


def _f(oparts):
    blk = 1024
    return pl.pallas_call(
        _f_body,
        grid=(NT // blk,),
        in_specs=[pl.BlockSpec((NC, blk, DIM), lambda i: (0, i, 0))],
        out_specs=pl.BlockSpec((blk, DIM), lambda i: (i, 0)),
        out_shape=jax.ShapeDtypeStruct((NT, DIM), jnp.float32),
    )(oparts)


def kernel(h_v, h_p, h_t, interacts_src, interacts_dst, agg_src, agg_dst,
           W_pi, W_q, W_r):
    i_src = interacts_src.astype(jnp.int32)
    i_dst = interacts_dst.astype(jnp.int32)
    a_src = agg_src.astype(jnp.int32)
    a_dst = agg_dst.astype(jnp.int32)
    w = W_pi.reshape(DIM)

    zf = jnp.zeros((NIP, DIM), jnp.float32)
    zm = jnp.zeros((NT, DIM), jnp.float32)
    zs = jnp.zeros((SR, DIM), jnp.float32)

    part, sparts = _p1(h_v, i_src, i_dst, w, zf)
    s_col = sparts.reshape(NW, NIP).sum(axis=0).reshape(NIP, 1)
    ft_item = _n(part, s_col)
    edge_ft, mparts, dparts = _b(ft_item, a_src, a_dst, zm, zs)
    deg_col = (dparts[0] + dparts[1]).reshape(SR * DIM)[:NT].reshape(NT, 1)
    f = _m1(mparts, deg_col, h_t, W_r)
    e2 = _m2(edge_ft, h_p, W_q)
    oparts = _c(edge_ft, e2, f, a_dst, zm)
    return _f(oparts)
